# baseline (speedup 1.0000x reference)
import functools

import jax
import jax.numpy as jnp
from jax import lax
from jax.experimental import pallas as pl
from jax.experimental.pallas import tpu as pltpu
from jax.experimental.pallas import tpu_sc as plsc

NC = 2
NS = 16
L = 16
NW = NC * NS

EPS = 1e-12


def _bcast(vec, i):
    idx = jnp.full((L, 1), i, jnp.int32)
    dn = lax.GatherDimensionNumbers(
        offset_dims=(), collapsed_slice_dims=(0,), start_index_map=(0,))
    return lax.gather(vec, idx, dn, slice_sizes=(1,),
                      mode=lax.GatherScatterMode.PROMISE_IN_BOUNDS)


def _rsqrt(x):
    i = plsc.bitcast(x, jnp.int32)
    i = jnp.int32(0x5F3759DF) - (i >> 1)
    y = plsc.bitcast(i, jnp.float32)
    for _ in range(3):
        y = y * (1.5 - 0.5 * x * y * y)
    return y


def _make_sc_kernel(n_tok, S, H, C):
    tok_per_w = n_tok // NW
    seq_per_w = tok_per_w // S
    n_chunks = S // C
    HV = H // L

    mesh = plsc.VectorSubcoreMesh(
        core_axis_name="c", subcore_axis_name="s", num_cores=NC, num_subcores=NS
    )

    @functools.partial(
        pl.kernel,
        out_type=jax.ShapeDtypeStruct((n_tok, H), jnp.float32),
        mesh=mesh,
        compiler_params=pltpu.CompilerParams(needs_layout_passes=False),
        scratch_types=[
            pltpu.VMEM((C, H), jnp.float32),
            pltpu.VMEM((C, H), jnp.float32),
            pltpu.VMEM((C, H), jnp.float32),
            pltpu.VMEM((C,), jnp.int32),
            pltpu.VMEM((C,), jnp.int32),
            pltpu.VMEM((2, H), jnp.float32),
            pltpu.VMEM((H,), jnp.float32),
            pltpu.VMEM((H,), jnp.float32),
            pltpu.VMEM((H,), jnp.float32),
            pltpu.SemaphoreType.DMA,
        ],
    )
    def sc_kernel(word_hbm, ids_hbm, tt_hbm, pos_hbm, type_hbm, gamma_hbm,
                  beta_hbm, out_hbm, wordbuf, posbuf, outbuf, idxbuf, ttbuf,
                  typebuf, dbuf, gammabuf, betabuf, sem):
        wid = lax.axis_index("s") * NC + lax.axis_index("c")
        base = wid * tok_per_w

        pltpu.sync_copy(type_hbm, typebuf)
        pltpu.sync_copy(gamma_hbm, gammabuf)
        pltpu.sync_copy(beta_hbm, betabuf)
        for j in range(HV):
            hs = pl.ds(j * L, L)
            dbuf[hs] = typebuf[1, hs] - typebuf[0, hs]

        def chunk_body(k, _):
            pltpu.sync_copy(pos_hbm.at[pl.ds(k * C, C)], posbuf)

            def seq_body(s, _):
                tok0 = base + s * S + k * C
                pltpu.sync_copy(ids_hbm.at[pl.ds(tok0, C)], idxbuf)
                pltpu.sync_copy(tt_hbm.at[pl.ds(tok0, C)], ttbuf)
                pltpu.async_copy(word_hbm.at[idxbuf], wordbuf, sem).wait()

                ttv = ttbuf[...]

                def tok_body(t, _):
                    ttf = _bcast(ttv, t).astype(jnp.float32)
                    acc_s = jnp.zeros((L,), jnp.float32)
                    acc_q = jnp.zeros((L,), jnp.float32)
                    for j in range(HV):
                        hs = pl.ds(j * L, L)
                        row = (wordbuf[t, hs] + posbuf[t, hs]
                               + typebuf[0, hs] + ttf * dbuf[hs])
                        acc_s = acc_s + row
                        acc_q = acc_q + row * row
                        outbuf[t, hs] = row
                    tot = _bcast(plsc.cumsum(acc_s), L - 1)
                    totq = _bcast(plsc.cumsum(acc_q), L - 1)
                    mean = tot * (1.0 / H)
                    var = totq * (1.0 / H) - mean * mean
                    inv = _rsqrt(var + EPS)
                    for j in range(HV):
                        hs = pl.ds(j * L, L)
                        a = gammabuf[hs] * inv
                        b = betabuf[hs] - mean * a
                        outbuf[t, hs] = outbuf[t, hs] * a + b
                    return 0

                lax.fori_loop(0, C, tok_body, 0)
                pltpu.sync_copy(outbuf, out_hbm.at[pl.ds(tok0, C)])
                return 0

            lax.fori_loop(0, seq_per_w, seq_body, 0)
            return 0

        lax.fori_loop(0, n_chunks, chunk_body, 0)

    return sc_kernel


@jax.jit
def kernel(input_ids, token_type_ids, word_table, pos_table, type_table,
           ln_gamma, ln_beta):
    B, S = input_ids.shape
    H = word_table.shape[1]
    n_tok = B * S
    C = 16
    ids32 = input_ids.reshape(-1).astype(jnp.int32)
    tt32 = token_type_ids.reshape(-1).astype(jnp.int32)
    sc = _make_sc_kernel(n_tok, S, H, C)
    out = sc(word_table, ids32, tt32, pos_table, type_table, ln_gamma, ln_beta)
    return out.reshape(B, S, H)
